# Initial kernel scaffold; baseline (speedup 1.0000x reference)
import functools

import jax
import jax.numpy as jnp
from jax import lax
from jax.experimental import pallas as pl
from jax.experimental.pallas import tpu as pltpu
from jax.experimental.pallas import tpu_sc as plsc

N = 100000
D = 128
H = 64
G = 512


_BLK = 2000


def _mlp_body(x_ref, w1_ref, b1_ref, w2_ref, b2_ref, o_ref):
    x = x_ref[...]
    h = jnp.dot(x, w1_ref[...], preferred_element_type=jnp.float32)
    h = h + b1_ref[...]
    h = h * jax.nn.sigmoid(h)
    r = jnp.sum(h * w2_ref[...], axis=1, keepdims=True) + b2_ref[...]
    o_ref[...] = r


def _mlp(x, W1, b1, W2, b2):
    grid = (N // _BLK,)
    return pl.pallas_call(
        _mlp_body,
        grid=grid,
        in_specs=[
            pl.BlockSpec((_BLK, D), lambda i: (i, 0)),
            pl.BlockSpec((D, H), lambda i: (0, 0)),
            pl.BlockSpec((1, H), lambda i: (0, 0)),
            pl.BlockSpec((1, H), lambda i: (0, 0)),
            pl.BlockSpec((1, 1), lambda i: (0, 0)),
        ],
        out_specs=pl.BlockSpec((_BLK, 1), lambda i: (i, 0)),
        out_shape=jax.ShapeDtypeStruct((N, 1), jnp.float32),
    )(x, W1, b1.reshape(1, H), W2.reshape(1, H), b2.reshape(1, 1))


_NS = 16
_NPAD = 102400
_ROWS = _NPAD // 128
_RPW = _ROWS // _NS


def _segsum_body(vals_hbm, idx_hbm, out_hbm, vals_v, idx_v, stage_v, acc_sp):
    s = lax.axis_index("s")
    base = s * _RPW
    pltpu.sync_copy(vals_hbm.at[pl.ds(base, _RPW)], vals_v)
    pltpu.sync_copy(idx_hbm.at[pl.ds(base, _RPW)], idx_v)

    @pl.when(s == 0)
    def _():
        for j in range(G // 16):
            stage_v[pl.ds(j * 16, 16)] = jnp.zeros((16,), jnp.float32)
        pltpu.sync_copy(stage_v, acc_sp)

    plsc.subcore_barrier()

    def step(j, carry):
        pltpu.sync_copy(vals_v.at[j], acc_sp.at[idx_v.at[j]], add=True)
        return carry

    lax.fori_loop(0, _RPW, step, 0)
    plsc.subcore_barrier()

    @pl.when(s == 0)
    def _():
        pltpu.sync_copy(acc_sp, out_hbm)


_segsum = pl.kernel(
    _segsum_body,
    out_type=jax.ShapeDtypeStruct((G,), jnp.float32),
    mesh=plsc.VectorSubcoreMesh(
        core_axis_name="c", subcore_axis_name="s", num_cores=1
    ),
    scratch_types=[
        pltpu.VMEM((_RPW, 128), jnp.float32),
        pltpu.VMEM((_RPW, 128), jnp.int32),
        pltpu.VMEM((G,), jnp.float32),
        pltpu.VMEM_SHARED((G,), jnp.float32),
    ],
)


def kernel(x_scalar, x_spherical, batch, W1, b1, W2, b2):
    res = _mlp(x_scalar, W1, b1, W2, b2)
    vals = jnp.pad(res.reshape(-1), (0, _NPAD - N)).reshape(_ROWS, 128)
    idx = jnp.pad(batch, (0, _NPAD - N)).reshape(_ROWS, 128)
    out = _segsum(vals, idx)
    return out.reshape(G, 1)


# speedup vs baseline: 1.3687x; 1.3687x over previous
import functools

import jax
import jax.numpy as jnp
from jax import lax
from jax.experimental import pallas as pl
from jax.experimental.pallas import tpu as pltpu
from jax.experimental.pallas import tpu_sc as plsc

N = 100000
D = 128
H = 64
G = 512


_BLK = 2000


def _mlp_body(x_ref, w1_ref, b1_ref, w2_ref, b2_ref, o_ref):
    x = x_ref[...]
    h = jnp.dot(x, w1_ref[...], preferred_element_type=jnp.float32)
    h = h + b1_ref[...]
    h = h * jax.nn.sigmoid(h)
    r = jnp.sum(h * w2_ref[...], axis=1, keepdims=True) + b2_ref[...]
    o_ref[...] = r


def _mlp(x, W1, b1, W2, b2):
    grid = (N // _BLK,)
    return pl.pallas_call(
        _mlp_body,
        grid=grid,
        in_specs=[
            pl.BlockSpec((_BLK, D), lambda i: (i, 0)),
            pl.BlockSpec((D, H), lambda i: (0, 0)),
            pl.BlockSpec((1, H), lambda i: (0, 0)),
            pl.BlockSpec((1, H), lambda i: (0, 0)),
            pl.BlockSpec((1, 1), lambda i: (0, 0)),
        ],
        out_specs=pl.BlockSpec((_BLK, 1), lambda i: (i, 0)),
        out_shape=jax.ShapeDtypeStruct((N, 1), jnp.float32),
    )(x, W1, b1.reshape(1, H), W2.reshape(1, H), b2.reshape(1, 1))


_NS = 16
_NPAD = 114688
_ROWS = _NPAD // 128
_RPW = _ROWS // _NS


def _segsum_body(vals_hbm, idx_hbm, out_hbm, vals_v, idx_v, stage_v, acc_sp):
    s = lax.axis_index("s")
    base = s * _RPW
    pltpu.sync_copy(vals_hbm.at[pl.ds(base, _RPW)], vals_v)
    pltpu.sync_copy(idx_hbm.at[pl.ds(base, _RPW)], idx_v)

    @pl.when(s == 0)
    def _():
        for j in range(G // 16):
            stage_v[pl.ds(j * 16, 16)] = jnp.zeros((16,), jnp.float32)
        pltpu.sync_copy(stage_v, acc_sp)

    plsc.subcore_barrier()

    def step(j, carry):
        pltpu.sync_copy(vals_v.at[j], acc_sp.at[idx_v.at[j]], add=True)
        return carry

    lax.fori_loop(0, _RPW, step, 0)
    plsc.subcore_barrier()

    @pl.when(s == 0)
    def _():
        pltpu.sync_copy(acc_sp, out_hbm)


@functools.cache
def _make_segsum():
    return pl.kernel(
        _segsum_body,
        out_type=jax.ShapeDtypeStruct((G,), jnp.float32),
        mesh=plsc.VectorSubcoreMesh(
            core_axis_name="c", subcore_axis_name="s",
            num_cores=1, num_subcores=_NS,
        ),
        scratch_types=[
            pltpu.VMEM((_RPW, 128), jnp.float32),
            pltpu.VMEM((_RPW, 128), jnp.int32),
            pltpu.VMEM((G,), jnp.float32),
            pltpu.VMEM_SHARED((G,), jnp.float32),
        ],
    )


def kernel(x_scalar, x_spherical, batch, W1, b1, W2, b2):
    res = _mlp(x_scalar, W1, b1, W2, b2)
    vals = jnp.pad(res.reshape(-1), (0, _NPAD - N)).reshape(_ROWS, 128)
    idx = jnp.pad(batch, (0, _NPAD - N)).reshape(_ROWS, 128)
    out = _make_segsum()(vals, idx)
    return out.reshape(G, 1)


# speedup vs baseline: 1.7982x; 1.3138x over previous
import functools

import jax
import jax.numpy as jnp
from jax import lax
from jax.experimental import pallas as pl
from jax.experimental.pallas import tpu as pltpu
from jax.experimental.pallas import tpu_sc as plsc

N = 100000
D = 128
H = 64
G = 512


_BLK = 2048
_GRID = -(-N // _BLK)
_OROWS = _GRID * _BLK // 128


def _mlp_body(x_ref, w1_ref, b1_ref, w2_ref, b2_ref, o_ref):
    i = pl.program_id(0)
    x = x_ref[...]
    h = jnp.dot(x, w1_ref[...], preferred_element_type=jnp.float32)
    h = h + b1_ref[...]
    h = h * jax.nn.sigmoid(h)
    r = jnp.sum(h * w2_ref[...], axis=1) + b2_ref[0, 0]
    r = r.reshape(_BLK // 128, 128)
    gid = i * _BLK + jax.lax.broadcasted_iota(jnp.int32, (_BLK // 128, 128), 0) * 128 \
        + jax.lax.broadcasted_iota(jnp.int32, (_BLK // 128, 128), 1)
    o_ref[...] = jnp.where(gid < N, r, 0.0)


def _mlp(x, W1, b1, W2, b2):
    return pl.pallas_call(
        _mlp_body,
        grid=(_GRID,),
        in_specs=[
            pl.BlockSpec((_BLK, D), lambda i: (i, 0)),
            pl.BlockSpec((D, H), lambda i: (0, 0)),
            pl.BlockSpec((1, H), lambda i: (0, 0)),
            pl.BlockSpec((1, H), lambda i: (0, 0)),
            pl.BlockSpec((1, 1), lambda i: (0, 0)),
        ],
        out_specs=pl.BlockSpec((_BLK // 128, 128), lambda i: (i, 0)),
        out_shape=jax.ShapeDtypeStruct((_OROWS, 128), jnp.float32),
    )(x, W1, b1.reshape(1, H), W2.reshape(1, H), b2.reshape(1, 1))


_NS = 16
_NPAD = 114688
_ROWS = _NPAD // 128
_RPW = _ROWS // _NS


def _segsum_body(vals_hbm, idx_hbm, out_hbm, vals_v, idx_v, stage_v, acc_sp):
    s = lax.axis_index("s")
    base = s * _RPW
    pltpu.sync_copy(vals_hbm.at[pl.ds(base, _RPW)], vals_v)
    pltpu.sync_copy(idx_hbm.at[pl.ds(base, _RPW)], idx_v)

    @pl.when(s == 0)
    def _():
        for j in range(G // 16):
            stage_v[pl.ds(j * 16, 16)] = jnp.zeros((16,), jnp.float32)
        pltpu.sync_copy(stage_v, acc_sp)

    plsc.subcore_barrier()

    def step(j, carry):
        pltpu.sync_copy(vals_v.at[j], acc_sp.at[idx_v.at[j]], add=True)
        return carry

    lax.fori_loop(0, _RPW, step, 0)
    plsc.subcore_barrier()

    @pl.when(s == 0)
    def _():
        pltpu.sync_copy(acc_sp, out_hbm)


@functools.cache
def _make_segsum():
    return pl.kernel(
        _segsum_body,
        out_type=jax.ShapeDtypeStruct((G,), jnp.float32),
        mesh=plsc.VectorSubcoreMesh(
            core_axis_name="c", subcore_axis_name="s",
            num_cores=1, num_subcores=_NS,
        ),
        scratch_types=[
            pltpu.VMEM((_RPW, 128), jnp.float32),
            pltpu.VMEM((_RPW, 128), jnp.int32),
            pltpu.VMEM((G,), jnp.float32),
            pltpu.VMEM_SHARED((G,), jnp.float32),
        ],
    )


def kernel(x_scalar, x_spherical, batch, W1, b1, W2, b2):
    res = _mlp(x_scalar, W1, b1, W2, b2)
    vals = jnp.pad(res, ((0, _ROWS - _OROWS), (0, 0)))
    idx = jnp.pad(batch, (0, _NPAD - N)).reshape(_ROWS, 128)
    out = _make_segsum()(vals, idx)
    return out.reshape(G, 1)


# speedup vs baseline: 1.8100x; 1.0066x over previous
import functools

import jax
import jax.numpy as jnp
from jax import lax
from jax.experimental import pallas as pl
from jax.experimental.pallas import tpu as pltpu
from jax.experimental.pallas import tpu_sc as plsc

N = 100000
D = 128
H = 64
G = 512


_BLK = 2048
_GRID = -(-N // _BLK)
_OROWS = _GRID * _BLK // 128


def _mlp_body(x_ref, w1_ref, b1_ref, w2_ref, b2_ref, o_ref):
    i = pl.program_id(0)
    x = x_ref[...]
    h = jnp.dot(x, w1_ref[...], preferred_element_type=jnp.float32)
    h = h + b1_ref[...]
    h = h * jax.nn.sigmoid(h)
    r = jnp.sum(h * w2_ref[...], axis=1) + b2_ref[0, 0]
    r = r.reshape(_BLK // 128, 128)
    gid = i * _BLK + jax.lax.broadcasted_iota(jnp.int32, (_BLK // 128, 128), 0) * 128 \
        + jax.lax.broadcasted_iota(jnp.int32, (_BLK // 128, 128), 1)
    o_ref[...] = jnp.where(gid < N, r, 0.0)


def _mlp(x, W1, b1, W2, b2):
    return pl.pallas_call(
        _mlp_body,
        grid=(_GRID,),
        in_specs=[
            pl.BlockSpec((_BLK, D), lambda i: (i, 0)),
            pl.BlockSpec((D, H), lambda i: (0, 0)),
            pl.BlockSpec((1, H), lambda i: (0, 0)),
            pl.BlockSpec((1, H), lambda i: (0, 0)),
            pl.BlockSpec((1, 1), lambda i: (0, 0)),
        ],
        out_specs=pl.BlockSpec((_BLK // 128, 128), lambda i: (i, 0)),
        out_shape=jax.ShapeDtypeStruct((_OROWS, 128), jnp.float32),
    )(x, W1, b1.reshape(1, H), W2.reshape(1, H), b2.reshape(1, 1))


_NS = 16
_NPAD = 114688
_ROWS = _NPAD // 128
_RPW = _ROWS // _NS


_CHUNK = 14


def _segsum_body(vals_hbm, idx_hbm, out_hbm, vals_v, idx_v, stage_v, acc_sp,
                 in_sem, sc_sem):
    s = lax.axis_index("s")
    base = s * _RPW
    vals_cp = pltpu.async_copy(vals_hbm.at[pl.ds(base, _RPW)], vals_v, in_sem)
    idx_cp = pltpu.async_copy(idx_hbm.at[pl.ds(base, _RPW)], idx_v, in_sem)

    @pl.when(s == 0)
    def _():
        for j in range(G // 16):
            stage_v[pl.ds(j * 16, 16)] = jnp.zeros((16,), jnp.float32)
        pltpu.sync_copy(stage_v, acc_sp)

    vals_cp.wait()
    idx_cp.wait()
    plsc.subcore_barrier()

    def chunk(c, carry):
        cps = [
            pltpu.async_copy(
                vals_v.at[c * _CHUNK + j],
                acc_sp.at[idx_v.at[c * _CHUNK + j]],
                sc_sem,
                add=True,
            )
            for j in range(_CHUNK)
        ]
        for cp in cps:
            cp.wait()
        return carry

    lax.fori_loop(0, _RPW // _CHUNK, chunk, 0)
    plsc.subcore_barrier()

    @pl.when(s == 0)
    def _():
        pltpu.sync_copy(acc_sp, out_hbm)


@functools.cache
def _make_segsum():
    return pl.kernel(
        _segsum_body,
        out_type=jax.ShapeDtypeStruct((G,), jnp.float32),
        mesh=plsc.VectorSubcoreMesh(
            core_axis_name="c", subcore_axis_name="s",
            num_cores=1, num_subcores=_NS,
        ),
        scratch_types=[
            pltpu.VMEM((_RPW, 128), jnp.float32),
            pltpu.VMEM((_RPW, 128), jnp.int32),
            pltpu.VMEM((G,), jnp.float32),
            pltpu.VMEM_SHARED((G,), jnp.float32),
            pltpu.SemaphoreType.DMA,
            pltpu.SemaphoreType.DMA,
        ],
    )


def kernel(x_scalar, x_spherical, batch, W1, b1, W2, b2):
    res = _mlp(x_scalar, W1, b1, W2, b2)
    vals = jnp.pad(res, ((0, _ROWS - _OROWS), (0, 0)))
    idx = jnp.pad(batch, (0, _NPAD - N)).reshape(_ROWS, 128)
    out = _make_segsum()(vals, idx)
    return out.reshape(G, 1)
